# Initial kernel scaffold; baseline (speedup 1.0000x reference)
import jax
import jax.numpy as jnp
from jax.experimental import pallas as pl


def kernel(x, w1, b1, w2, b2):
    raise NotImplementedError("write your pallas kernel here")


# speedup vs baseline: 1.2173x; 1.2173x over previous
import functools

import jax
import jax.numpy as jnp
from jax.experimental import pallas as pl
from jax.experimental.pallas import tpu as pltpu

_VMEM_CAP = 48 << 20


def _se_kernel(x_ref, w1t_ref, b1_ref, w2t_ref, b2_ref, o_ref, *, inv_hw):
    x = x_ref[...]
    m = jnp.sum(x, axis=2, dtype=jnp.float32) * inv_hw
    z = jnp.dot(m, w1t_ref[...],
                preferred_element_type=jnp.float32) + b1_ref[...]
    z = jnp.clip(z, 0.0, 6.0)
    e = jnp.dot(z, w2t_ref[...],
                preferred_element_type=jnp.float32) + b2_ref[...]
    e = jax.nn.sigmoid(e)
    o_ref[...] = x * e[:, :, None].astype(x.dtype)


def _pick_batch_tile(n, slab_bytes):
    budget = 8 << 20
    best = 1
    for b in range(1, n + 1):
        if n % b:
            continue
        if b * slab_bytes <= budget and n // b >= 2:
            best = b
    return best


def kernel(x, w1, b1, w2, b2):
    N, C, H, W = x.shape
    HW = H * W
    C_mid = w1.shape[0]
    x_flat = x.reshape(N, C, HW)

    w1t = jnp.asarray(w1, jnp.float32).T
    b1r = jnp.asarray(b1, jnp.float32).reshape(1, C_mid)
    w2t = jnp.asarray(w2, jnp.float32).T
    b2r = jnp.asarray(b2, jnp.float32).reshape(1, C)

    slab_bytes = C * HW * x.dtype.itemsize
    B = _pick_batch_tile(N, slab_bytes)
    grid_n = N // B

    vmem = int(min(_VMEM_CAP, 4 * B * slab_bytes + (2 << 20)))

    out = pl.pallas_call(
        functools.partial(_se_kernel, inv_hw=1.0 / HW),
        out_shape=jax.ShapeDtypeStruct((N, C, HW), x.dtype),
        grid_spec=pltpu.PrefetchScalarGridSpec(
            num_scalar_prefetch=0,
            grid=(grid_n,),
            in_specs=[
                pl.BlockSpec((B, C, HW), lambda n: (n, 0, 0)),
                pl.BlockSpec((C, C_mid), lambda n: (0, 0)),
                pl.BlockSpec((1, C_mid), lambda n: (0, 0)),
                pl.BlockSpec((C_mid, C), lambda n: (0, 0)),
                pl.BlockSpec((1, C), lambda n: (0, 0)),
            ],
            out_specs=pl.BlockSpec((B, C, HW), lambda n: (n, 0, 0)),
        ),
        compiler_params=pltpu.CompilerParams(
            dimension_semantics=("parallel",),
            vmem_limit_bytes=vmem),
    )(x_flat, w1t, b1r, w2t, b2r)
    return out.reshape(N, C, H, W)
